# baseline (speedup 1.0000x reference)
import functools

import jax
import jax.numpy as jnp
from jax import lax
from jax.experimental import pallas as pl
from jax.experimental.pallas import tpu as pltpu
from jax.experimental.pallas import tpu_sc as plsc

D_X = 4
D_Y = 4
N_PART = 512
BATCH = 8
T_EXT = 16
SIG_X = 0.5
SIG_Y = 0.5

_F32 = jnp.float32
_CHUNK = 128
_NCHUNK = N_PART // _CHUNK


def _gather_chunks(planes, idx):
    wi = jax.lax.bitwise_and(idx, _CHUNK - 1)
    ch = jax.lax.shift_right_logical(idx, 7)
    out = jnp.take_along_axis(planes[0], wi, axis=1)
    for c in range(1, _NCHUNK):
        out = jnp.where(ch == c, jnp.take_along_axis(planes[c], wi, axis=1),
                        out)
    return out


def _resample_gather_body(cum_ref, u_ref, x_ref, out_ref):
    n_f = jax.lax.broadcasted_iota(jnp.int32, (BATCH, N_PART), 1).astype(_F32)
    p = (n_f + u_ref[...]) * (1.0 / N_PART)
    cum_planes = [cum_ref[:, c * _CHUNK:(c + 1) * _CHUNK]
                  for c in range(_NCHUNK)]
    lo = jnp.zeros((BATCH, N_PART), jnp.int32)
    hi = jnp.full((BATCH, N_PART), N_PART, jnp.int32)
    for _ in range(10):
        mid = jnp.minimum(jax.lax.shift_right_logical(lo + hi, 1), N_PART - 1)
        cv = _gather_chunks(cum_planes, mid)
        pred = cv < p
        lo = jnp.where(pred, mid + 1, lo)
        hi = jnp.where(pred, hi, mid)
    idx = jnp.minimum(lo, N_PART - 1)
    for d in range(D_X):
        x_d = x_ref[d]
        planes = [x_d[:, c * _CHUNK:(c + 1) * _CHUNK] for c in range(_NCHUNK)]
        out_ref[d] = _gather_chunks(planes, idx)


def _resample_gather(logw, u, x):
    w = jnp.exp(logw)
    cum = jnp.cumsum(w, axis=1)
    x_dmaj = x.transpose(2, 0, 1)
    x_res = pl.pallas_call(
        _resample_gather_body,
        in_specs=[
            pl.BlockSpec(memory_space=pltpu.VMEM),
            pl.BlockSpec(memory_space=pltpu.VMEM),
            pl.BlockSpec(memory_space=pltpu.VMEM),
        ],
        out_specs=pl.BlockSpec(memory_space=pltpu.VMEM),
        out_shape=jax.ShapeDtypeStruct((D_X, BATCH, N_PART), _F32),
    )(cum, u[:, None], x_dmaj)
    return x_res.transpose(1, 2, 0)


def _resample_gather_sc(logw, u, x):
    w = jnp.exp(logw)
    cum = jnp.cumsum(w, axis=1)
    x_dmaj = x.transpose(2, 0, 1)
    u16 = jnp.broadcast_to(u[:, None], (BATCH, 16))

    mesh = plsc.VectorSubcoreMesh(core_axis_name="c", subcore_axis_name="s")

    @functools.partial(
        pl.kernel, mesh=mesh,
        compiler_params=pltpu.CompilerParams(
            use_tc_tiling_on_sc=True, needs_layout_passes=False),
        out_type=jax.ShapeDtypeStruct((D_X, BATCH, N_PART), _F32),
        scratch_types=[
            pltpu.VMEM((N_PART,), _F32),
            pltpu.VMEM((D_X, N_PART), _F32),
            pltpu.VMEM((16,), _F32),
            pltpu.VMEM((D_X, _CHUNK), _F32),
        ],
    )
    def sc_kernel(cum_hbm, u_hbm, x_hbm, out_hbm, cum_v, x_v, u_v, o_v):
        wid = lax.axis_index("s") * 2 + lax.axis_index("c")
        b = wid // D_X
        n0 = (wid % D_X) * _CHUNK
        pltpu.sync_copy(cum_hbm.at[b], cum_v)
        pltpu.sync_copy(u_hbm.at[b], u_v)
        for d in range(D_X):
            pltpu.sync_copy(x_hbm.at[d].at[b], x_v.at[d])
        uvec = u_v[...]
        n0_f = lax.convert_element_type(n0, _F32)
        for g in range(N_PART // D_X // 16):
            n_f = lax.broadcasted_iota(jnp.int32, (16,), 0).astype(_F32) + (
                n0_f + float(g * 16))
            p = (n_f + uvec) * (1.0 / N_PART)
            lo = jnp.zeros((16,), jnp.int32)
            hi = jnp.full((16,), N_PART, jnp.int32)
            for _ in range(10):
                mid = jnp.minimum(
                    jax.lax.shift_right_logical(lo + hi, 1), N_PART - 1)
                cv = plsc.load_gather(cum_v, [mid])
                pred = cv < p
                lo = jnp.where(pred, mid + 1, lo)
                hi = jnp.where(pred, hi, mid)
            idx = jnp.minimum(lo, N_PART - 1)
            for d in range(D_X):
                o_v[d, pl.ds(g * 16, 16)] = plsc.load_gather(
                    x_v, [jnp.full((16,), d, jnp.int32), idx])
        for d in range(D_X):
            pltpu.sync_copy(o_v.at[d], out_hbm.at[d].at[b].at[pl.ds(n0, _CHUNK)])

    x_res = sc_kernel(cum, u16, x_dmaj)
    return x_res.transpose(1, 2, 0)


def _log_obs_density(x, y, C):
    mean = jnp.einsum('od,bnd->bno', C, x)
    diff = (y[:, None, :] - mean) / SIG_Y
    return (-0.5 * jnp.sum(diff * diff, axis=-1)
            - 0.5 * D_Y * jnp.log(2.0 * jnp.pi * SIG_Y ** 2))


def kernel(observation, A, C, init_noise, step_noise, resample_u):
    x = init_noise
    lw_un = _log_obs_density(x, observation[0], C)
    logw = lw_un - jax.nn.logsumexp(lw_un, axis=1, keepdims=True)
    outputs = [None] * (T_EXT + 1)
    for t in range(1, T_EXT + 1):
        outputs[t - 1] = jnp.einsum('bn,bnd->bd', jnp.exp(logw), x)
        x_res = _resample_gather_sc(jax.lax.stop_gradient(logw),
                                    resample_u[t - 1], x)
        x_new = jnp.einsum('od,bnd->bno', A, x_res) + SIG_X * step_noise[t - 1]
        lw_un = _log_obs_density(x_new, observation[t], C)
        logw = lw_un - jax.nn.logsumexp(lw_un, axis=1, keepdims=True)
        x = x_new
    outputs[T_EXT] = jnp.einsum('bn,bnd->bd', jnp.exp(logw), x)
    return jnp.stack(outputs, axis=0)


# speedup vs baseline: 1.2571x; 1.2571x over previous
import jax
import jax.numpy as jnp
from jax.experimental import pallas as pl
from jax.experimental.pallas import tpu as pltpu

D_X = 4
D_Y = 4
N_PART = 512
BATCH = 8
T_EXT = 16
SIG_X = 0.5
SIG_Y = 0.5

_F32 = jnp.float32
_CHUNK = 128
_NCHUNK = N_PART // _CHUNK


def _gather_chunks(planes, idx):
    wi = jax.lax.bitwise_and(idx, _CHUNK - 1)
    ch = jax.lax.shift_right_logical(idx, 7)
    out = jnp.take_along_axis(planes[0], wi, axis=1)
    for c in range(1, _NCHUNK):
        out = jnp.where(ch == c, jnp.take_along_axis(planes[c], wi, axis=1),
                        out)
    return out


def _resample_gather_body(cum_ref, u_ref, x_ref, out_ref):
    n_f = jax.lax.broadcasted_iota(jnp.int32, (BATCH, N_PART), 1).astype(_F32)
    p = (n_f + u_ref[...]) * (1.0 / N_PART)
    cum_planes = [cum_ref[:, c * _CHUNK:(c + 1) * _CHUNK]
                  for c in range(_NCHUNK)]
    lo = jnp.zeros((BATCH, N_PART), jnp.int32)
    hi = jnp.full((BATCH, N_PART), N_PART, jnp.int32)
    for _ in range(10):
        mid = jnp.minimum(jax.lax.shift_right_logical(lo + hi, 1), N_PART - 1)
        cv = _gather_chunks(cum_planes, mid)
        pred = cv < p
        lo = jnp.where(pred, mid + 1, lo)
        hi = jnp.where(pred, hi, mid)
    idx = jnp.minimum(lo, N_PART - 1)
    for d in range(D_X):
        x_d = x_ref[d]
        planes = [x_d[:, c * _CHUNK:(c + 1) * _CHUNK] for c in range(_NCHUNK)]
        out_ref[d] = _gather_chunks(planes, idx)


def _resample_gather(logw, u, x):
    w = jnp.exp(logw)
    cum = jnp.cumsum(w, axis=1)
    x_dmaj = x.transpose(2, 0, 1)
    x_res = pl.pallas_call(
        _resample_gather_body,
        in_specs=[
            pl.BlockSpec(memory_space=pltpu.VMEM),
            pl.BlockSpec(memory_space=pltpu.VMEM),
            pl.BlockSpec(memory_space=pltpu.VMEM),
        ],
        out_specs=pl.BlockSpec(memory_space=pltpu.VMEM),
        out_shape=jax.ShapeDtypeStruct((D_X, BATCH, N_PART), _F32),
    )(cum, u[:, None], x_dmaj)
    return x_res.transpose(1, 2, 0)


def _normalize_body(lw_ref, out_ref):
    a = lw_ref[...]
    amax = jnp.max(a, axis=1, keepdims=True)
    amax = jnp.where(jnp.isfinite(amax), amax, 0.0)
    s = jnp.sum(jnp.exp(a - amax), axis=1, keepdims=True)
    out_ref[...] = a - (jnp.log(s) + amax)


def _normalize(lw_un):
    return pl.pallas_call(
        _normalize_body,
        out_shape=jax.ShapeDtypeStruct((BATCH, N_PART), _F32),
    )(lw_un)


def _log_obs_density(x, y, C):
    mean = jnp.einsum('od,bnd->bno', C, x)
    diff = (y[:, None, :] - mean) / SIG_Y
    return (-0.5 * jnp.sum(diff * diff, axis=-1)
            - 0.5 * D_Y * jnp.log(2.0 * jnp.pi * SIG_Y ** 2))


def kernel(observation, A, C, init_noise, step_noise, resample_u):
    x = init_noise
    logw = _normalize(_log_obs_density(x, observation[0], C))
    outputs = [None] * (T_EXT + 1)
    for t in range(1, T_EXT + 1):
        outputs[t - 1] = jnp.einsum('bn,bnd->bd', jnp.exp(logw), x)
        x_res = _resample_gather(jax.lax.stop_gradient(logw),
                                 resample_u[t - 1], x)
        x_new = jnp.einsum('od,bnd->bno', A, x_res) + SIG_X * step_noise[t - 1]
        logw = _normalize(_log_obs_density(x_new, observation[t], C))
        x = x_new
    outputs[T_EXT] = jnp.einsum('bn,bnd->bd', jnp.exp(logw), x)
    return jnp.stack(outputs, axis=0)


# speedup vs baseline: 1.7816x; 1.4172x over previous
import jax
import jax.numpy as jnp
from jax.experimental import pallas as pl
from jax.experimental.pallas import tpu as pltpu

D_X = 4
D_Y = 4
N_PART = 512
BATCH = 8
T_EXT = 16
SIG_X = 0.5
SIG_Y = 0.5

_F32 = jnp.float32
_CHUNK = 128
_NCHUNK = N_PART // _CHUNK


def _gather_chunks(planes, idx):
    wi = jax.lax.bitwise_and(idx, _CHUNK - 1)
    ch = jax.lax.shift_right_logical(idx, 7)
    out = jnp.take_along_axis(planes[0], wi, axis=1)
    for c in range(1, _NCHUNK):
        out = jnp.where(ch == c, jnp.take_along_axis(planes[c], wi, axis=1),
                        out)
    return out


def _resample_gather_body(cum_ref, u_ref, x_ref, out_ref):
    n_f = jax.lax.broadcasted_iota(jnp.int32, (BATCH, N_PART), 1).astype(_F32)
    p = (n_f + u_ref[...]) * (1.0 / N_PART)
    cum_planes = [cum_ref[:, c * _CHUNK:(c + 1) * _CHUNK]
                  for c in range(_NCHUNK)]
    lo = jnp.zeros((BATCH, N_PART), jnp.int32)
    hi = jnp.full((BATCH, N_PART), N_PART, jnp.int32)
    for _ in range(10):
        mid = jnp.minimum(jax.lax.shift_right_logical(lo + hi, 1), N_PART - 1)
        cv = _gather_chunks(cum_planes, mid)
        pred = cv < p
        lo = jnp.where(pred, mid + 1, lo)
        hi = jnp.where(pred, hi, mid)
    idx = jnp.minimum(lo, N_PART - 1)
    for d in range(D_X):
        x_d = x_ref[d]
        planes = [x_d[:, c * _CHUNK:(c + 1) * _CHUNK] for c in range(_NCHUNK)]
        out_ref[d] = _gather_chunks(planes, idx)


def _resample_gather(logw, u, x):
    w = jnp.exp(logw)
    cum = jnp.cumsum(w, axis=1)
    x_dmaj = x.transpose(2, 0, 1)
    x_res = pl.pallas_call(
        _resample_gather_body,
        in_specs=[
            pl.BlockSpec(memory_space=pltpu.VMEM),
            pl.BlockSpec(memory_space=pltpu.VMEM),
            pl.BlockSpec(memory_space=pltpu.VMEM),
        ],
        out_specs=pl.BlockSpec(memory_space=pltpu.VMEM),
        out_shape=jax.ShapeDtypeStruct((D_X, BATCH, N_PART), _F32),
    )(cum, u[:, None], x_dmaj)
    return x_res.transpose(1, 2, 0)


def _log_obs_density(x, y, C):
    mean = jnp.einsum('od,bnd->bno', C, x)
    diff = (y[:, None, :] - mean) / SIG_Y
    return (-0.5 * jnp.sum(diff * diff, axis=-1)
            - 0.5 * D_Y * jnp.log(2.0 * jnp.pi * SIG_Y ** 2))


def kernel(observation, A, C, init_noise, step_noise, resample_u):
    x = init_noise
    lw_un = _log_obs_density(x, observation[0], C)
    logw = lw_un - jax.nn.logsumexp(lw_un, axis=1, keepdims=True)
    outputs = [None] * (T_EXT + 1)
    for t in range(1, T_EXT + 1):
        outputs[t - 1] = jnp.einsum('bn,bnd->bd', jnp.exp(logw), x)
        x_res = _resample_gather(jax.lax.stop_gradient(logw),
                                 resample_u[t - 1], x)
        x_new = jnp.einsum('od,bnd->bno', A, x_res) + SIG_X * step_noise[t - 1]
        lw_un = _log_obs_density(x_new, observation[t], C)
        logw = lw_un - jax.nn.logsumexp(lw_un, axis=1, keepdims=True)
        x = x_new
    outputs[T_EXT] = jnp.einsum('bn,bnd->bd', jnp.exp(logw), x)
    return jnp.stack(outputs, axis=0)
